# Initial kernel scaffold; baseline (speedup 1.0000x reference)
import jax
import jax.numpy as jnp
from jax.experimental import pallas as pl


def kernel(points, W_edge, b_edge, Wih_f, Whh_f, b_f, Wih_b, Whh_b, b_b, W1, b1, W2, b2):
    raise NotImplementedError("write your pallas kernel here")


# speedup vs baseline: 12.9647x; 12.9647x over previous
import functools

import jax
import jax.numpy as jnp
from jax import lax
from jax.experimental import pallas as pl
from jax.experimental.pallas import tpu as pltpu
from jax.experimental.pallas import tpu_sc as plsc

S, B, P, K, E, H = 16, 4, 2048, 16, 256, 256
G = S * B
RB = 512
NRB = P // RB


def _topk_body(x_ref, idx_ref, score_ref):
    r = pl.program_id(1)
    x2 = x_ref[0]
    xr = x_ref[0, pl.ds(r * RB, RB), :]
    dn = (((1,), (1,)), ((), ()))
    g = lax.dot_general(xr, x2, dn)
    xx = x2 * x2
    sq_row = lax.dot_general(jnp.ones((1, 2), jnp.float32), xx, dn)
    sq_col = jnp.sum(xr * xr, axis=1, keepdims=True)
    ri = lax.broadcasted_iota(jnp.int32, (RB, P), 0) + r * RB
    ci = lax.broadcasted_iota(jnp.int32, (RB, P), 1)
    d2 = sq_col + sq_row - 2.0 * g
    d2 = d2 + jnp.where(ri == ci, jnp.float32(1e10), jnp.float32(0.0))
    score_ref[...] = d2
    for k in range(K):
        s = score_ref[...]
        m = jnp.min(s, axis=1, keepdims=True)
        cand = jnp.where(s == m, ci, jnp.int32(P))
        a = jnp.min(cand, axis=1, keepdims=True)
        idx_ref[0, :, k:k + 1] = a
        score_ref[...] = jnp.where(ci == a, jnp.float32(3e38), s)


def _topk(pts64):
    return pl.pallas_call(
        _topk_body,
        grid=(G, NRB),
        in_specs=[pl.BlockSpec((1, P, 2), lambda g, r: (g, 0, 0))],
        out_specs=pl.BlockSpec((1, RB, K), lambda g, r: (g, r, 0)),
        out_shape=jax.ShapeDtypeStruct((G, P, K), jnp.int32),
        scratch_shapes=[pltpu.VMEM((RB, P), jnp.float32)],
    )(pts64)


GPW = G // 32
IC = 8192
NIC = (P * K) // IC


def _sc_gather(xsflat, ysflat, idxabs):
    mesh = plsc.VectorSubcoreMesh(core_axis_name="c", subcore_axis_name="s")

    @functools.partial(
        pl.kernel,
        mesh=mesh,
        out_type=[jax.ShapeDtypeStruct((G, P * K), jnp.float32),
                  jax.ShapeDtypeStruct((G, P * K), jnp.float32)],
        scratch_types=[pltpu.VMEM((IC,), jnp.int32),
                       pltpu.VMEM((IC,), jnp.float32),
                       pltpu.VMEM((IC,), jnp.float32),
                       pltpu.SemaphoreType.DMA,
                       pltpu.SemaphoreType.DMA],
    )
    def k(xs_hbm, ys_hbm, idx_hbm, xn_hbm, yn_hbm, iv, xo, yo, s1, s2):
        wid = lax.axis_index("s") * 2 + lax.axis_index("c")
        for i in range(GPW):
            gg = wid * GPW + i
            for c in range(NIC):
                pltpu.sync_copy(idx_hbm.at[gg, pl.ds(c * IC, IC)], iv)
                cx = pltpu.async_copy(xs_hbm.at[iv], xo, s1)
                cy = pltpu.async_copy(ys_hbm.at[iv], yo, s2)
                cx.wait()
                cy.wait()
                pltpu.sync_copy(xo, xn_hbm.at[gg, pl.ds(c * IC, IC)])
                pltpu.sync_copy(yo, yn_hbm.at[gg, pl.ds(c * IC, IC)])

    return k(xsflat, ysflat, idxabs)


def _edge_body(x_ref, xn_ref, yn_ref, a2_ref, c2_ref, b_ref, out_ref):
    x2 = x_ref[0]
    xn = xn_ref[0]
    yn = yn_ref[0]
    a2 = a2_ref[...]
    c2 = c2_ref[...]
    u = x2[:, 0:1] * a2[0:1, :] + x2[:, 1:2] * a2[1:2, :]
    mu = xn[:, 0:1] * c2[0:1, :] + yn[:, 0:1] * c2[1:2, :]
    for k in range(1, K):
        zk = xn[:, k:k + 1] * c2[0:1, :] + yn[:, k:k + 1] * c2[1:2, :]
        mu = jnp.maximum(mu, zk)
    emb = jnp.max(u + mu, axis=0, keepdims=True) + b_ref[...]
    out_ref[0] = jnp.maximum(emb, jnp.float32(0.0))


def _edge(pts64, xn3, yn3, a2, c2, b_row):
    return pl.pallas_call(
        _edge_body,
        grid=(G,),
        in_specs=[pl.BlockSpec((1, P, 2), lambda g: (g, 0, 0)),
                  pl.BlockSpec((1, P, K), lambda g: (g, 0, 0)),
                  pl.BlockSpec((1, P, K), lambda g: (g, 0, 0)),
                  pl.BlockSpec((2, E), lambda g: (0, 0)),
                  pl.BlockSpec((2, E), lambda g: (0, 0)),
                  pl.BlockSpec((1, E), lambda g: (0, 0))],
        out_specs=pl.BlockSpec((1, 1, E), lambda g: (g, 0, 0)),
        out_shape=jax.ShapeDtypeStruct((G, 1, E), jnp.float32),
    )(pts64, xn3, yn3, a2, c2, b_row)


def _lstm_body(emb_ref, wihf_ref, whhf_ref, bf_ref, wihb_ref, whhb_ref,
               bb_ref, w1_ref, b1_ref, w2_ref, b2_ref, out_ref):
    dn = (((1,), (1,)), ((), ()))

    def run_dir(wih, whh, b, reverse):
        def step(t, carry):
            h, c = carry
            s = jnp.where(reverse, S - 1 - t, t)
            x_t = emb_ref[pl.ds(s * 8, 8), :]
            gts = (lax.dot_general(x_t, wih, dn) +
                   lax.dot_general(h, whh, dn) + b)
            i = jax.nn.sigmoid(gts[:, 0:H])
            f = jax.nn.sigmoid(gts[:, H:2 * H])
            gg = jnp.tanh(gts[:, 2 * H:3 * H])
            o = jax.nn.sigmoid(gts[:, 3 * H:4 * H])
            c = f * c + i * gg
            h = o * jnp.tanh(c)
            return (h, c)

        z = jnp.zeros((8, H), jnp.float32)
        h, _ = lax.fori_loop(0, S, step, (z, z))
        return h

    hf = run_dir(wihf_ref[...], whhf_ref[...], bf_ref[...], False)
    hb = run_dir(wihb_ref[...], whhb_ref[...], bb_ref[...], True)
    gcat = jnp.concatenate([hf, hb], axis=1)
    z = jnp.maximum(lax.dot_general(gcat, w1_ref[...],
                                    (((1,), (0,)), ((), ()))) + b1_ref[...],
                    jnp.float32(0.0))
    res = lax.dot_general(z, w2_ref[...],
                          (((1,), (0,)), ((), ()))) + b2_ref[...]
    out_ref[...] = jnp.broadcast_to(res, (8, 128))


def _lstm(emb128, wihf, whhf, bf, wihb, whhb, bb, w1, b1, w2, b2):
    full = lambda shp: pl.BlockSpec(shp, lambda: tuple(0 for _ in shp))
    return pl.pallas_call(
        _lstm_body,
        in_specs=[full((S * 8, E)), full((4 * H, E)), full((4 * H, H)),
                  full((1, 4 * H)), full((4 * H, E)), full((4 * H, H)),
                  full((1, 4 * H)), full((2 * H, 128)), full((1, 128)),
                  full((128, 1)), full((1, 1))],
        out_specs=full((8, 128)),
        out_shape=jax.ShapeDtypeStruct((8, 128), jnp.float32),
    )(emb128, wihf, whhf, bf, wihb, whhb, bb, w1, b1, w2, b2)


@jax.jit
def kernel(points, W_edge, b_edge, Wih_f, Whh_f, b_f, Wih_b, Whh_b, b_b,
           W1, b1, W2, b2):
    pts64 = points.reshape(G, P, 2)
    xs = pts64[:, :, 0]
    ys = pts64[:, :, 1]

    idx = _topk(pts64)
    idxabs = (idx.reshape(G, P * K) +
              (jnp.arange(G, dtype=jnp.int32) * P)[:, None])
    xn, yn = _sc_gather(xs.reshape(G * P), ys.reshape(G * P), idxabs)
    a2 = W_edge[0:2] - W_edge[2:4]
    c2 = W_edge[2:4]
    emb = _edge(pts64, xn.reshape(G, P, K), yn.reshape(G, P, K),
                a2, c2, b_edge.reshape(1, E))
    emb = emb.reshape(G, E)

    emb128 = jnp.pad(emb.reshape(S, B, E), ((0, 0), (0, 4), (0, 0)))
    emb128 = emb128.reshape(S * 8, E)
    out = _lstm(emb128, Wih_f, Whh_f, b_f.reshape(1, 4 * H),
                Wih_b, Whh_b, b_b.reshape(1, 4 * H),
                W1, b1.reshape(1, 128), W2, b2.reshape(1, 1))
    return out[0:B, 0]
